# baseline (speedup 1.0000x reference)
import jax
import jax.numpy as jnp
from jax.experimental import pallas as pl
from jax.experimental.pallas import tpu as pltpu

_BM = 512
_SLOTS = 4
_CHUNKS = 8
_CR = _BM // _CHUNKS


def _chunk_copy(h_hbm, scratch, sems, block, slot, c):
    return pltpu.make_async_copy(
        h_hbm.at[pl.ds(block * _BM + c * _CR, _CR), :],
        scratch.at[slot, pl.ds(c * _CR, _CR), :],
        sems.at[slot, c],
    )


def _start_block(h_hbm, scratch, sems, block, slot):
    for c in range(_CHUNKS):
        _chunk_copy(h_hbm, scratch, sems, block, slot, c).start()


def _wait_block(h_hbm, scratch, sems, block, slot):
    for c in range(_CHUNKS):
        _chunk_copy(h_hbm, scratch, sems, block, slot, c).wait()


def _matmul_block(h_hbm, wq_ref, out_ref, scratch, sems):
    i = pl.program_id(0)
    n = pl.num_programs(0)

    @pl.when(i == 0)
    def _prologue():
        for s in range(_SLOTS):
            _start_block(h_hbm, scratch, sems, s, s)

    slot = jax.lax.rem(i, _SLOTS)
    _wait_block(h_hbm, scratch, sems, i, slot)
    out_ref[...] = jax.lax.dot_general(
        scratch[slot],
        wq_ref[...],
        dimension_numbers=(((1,), (1,)), ((), ())),
        preferred_element_type=jnp.float32,
        precision=jax.lax.Precision.DEFAULT,
    )

    @pl.when(i + _SLOTS < n)
    def _refill():
        _start_block(h_hbm, scratch, sems, i + _SLOTS, slot)


@jax.jit
def kernel(h, Wq, Wn):
    del Wn
    m, d = h.shape
    e = Wq.shape[0]
    grid = (m // _BM,)
    return pl.pallas_call(
        _matmul_block,
        grid=grid,
        in_specs=[
            pl.BlockSpec(memory_space=pltpu.MemorySpace.HBM),
            pl.BlockSpec((e, d), lambda i: (0, 0)),
        ],
        out_specs=pl.BlockSpec((_BM, e), lambda i: (i, 0)),
        out_shape=jax.ShapeDtypeStruct((m, e), jnp.float32),
        scratch_shapes=[
            pltpu.VMEM((_SLOTS, _BM, d), jnp.float32),
            pltpu.SemaphoreType.DMA((_SLOTS, _CHUNKS)),
        ],
        compiler_params=pltpu.CompilerParams(
            dimension_semantics=("arbitrary",),
        ),
    )(h, Wq)


# speedup vs baseline: 1.0144x; 1.0144x over previous
import jax
import jax.numpy as jnp
from jax.experimental import pallas as pl
from jax.experimental.pallas import tpu as pltpu

_BM = 512


def _matmul_block(h_ref, wq_ref, out_ref):
    out_ref[...] = jax.lax.dot_general(
        h_ref[...],
        wq_ref[...],
        dimension_numbers=(((1,), (1,)), ((), ())),
        preferred_element_type=jnp.float32,
        precision=jax.lax.Precision.DEFAULT,
    )


@jax.jit
def kernel(h, Wq, Wn):
    del Wn
    m, d = h.shape
    e = Wq.shape[0]
    grid = (m // _BM,)
    return pl.pallas_call(
        _matmul_block,
        grid=grid,
        in_specs=[
            pl.BlockSpec((_BM, d), lambda i: (i, 0)),
            pl.BlockSpec((e, d), lambda i: (0, 0)),
        ],
        out_specs=pl.BlockSpec((_BM, e), lambda i: (i, 0)),
        out_shape=jax.ShapeDtypeStruct((m, e), jnp.float32),
        compiler_params=pltpu.CompilerParams(
            dimension_semantics=("arbitrary",),
        ),
    )(h, Wq)
